# Initial kernel scaffold; baseline (speedup 1.0000x reference)
import jax
import jax.numpy as jnp
from jax.experimental import pallas as pl


def kernel(edge_params, velocities, iter_idx):
    raise NotImplementedError("write your pallas kernel here")


# speedup vs baseline: 1.4163x; 1.4163x over previous
import functools

import jax
import jax.numpy as jnp
import numpy as np
from jax import lax
from jax.experimental import pallas as pl
from jax.experimental.pallas import tpu as pltpu

N = 16384
H = 2048
W = 2048
L = 32

PAD = 16
NDUMP = 2
HP = H + 2 * PAD
HC = HP + NDUMP
BLK = 128
GRID = H // BLK

import ml_dtypes

_x = (np.arange(11, dtype=np.float32) - 5.0).astype(np.float32)
_k = np.exp(np.float32(-0.5) * (_x / np.float32(2.0)) ** 2, dtype=np.float32)
GW = ((_k / _k.sum(dtype=np.float32)).astype(np.float32)
      .astype(ml_dtypes.bfloat16).astype(np.float32))
C_NOM = 1.0
C_MAX = float(np.float32(1.02).astype(ml_dtypes.bfloat16))
C_MIN = float(np.float32(0.98).astype(ml_dtypes.bfloat16))


def _blur_body(canvas_ref, mask_ref, nom_ref, mx_ref, mn_ref):
    i = pl.program_id(0)
    p0 = i * BLK + PAD
    win = canvas_ref[pl.ds(p0 - 8, BLK + 16), :]
    mask_ref[...] = lax.slice(win, (8, 0), (8 + BLK, W))

    acc = float(GW[0]) * lax.slice(win, (3, 0), (3 + BLK, W))
    for k in range(1, 11):
        acc += float(GW[k]) * lax.slice(win, (3 + k, 0), (3 + k + BLK, W))

    z = jnp.zeros((BLK, 8), jnp.float32)
    for c, out_ref in ((C_NOM, nom_ref), (C_MAX, mx_ref), (C_MIN, mn_ref)):
        u = (c * acc if c != 1.0 else acc)
        u = u.astype(jnp.bfloat16).astype(jnp.float32)
        padded = jnp.concatenate([z, u, z], axis=1)
        b = float(GW[0]) * lax.slice(padded, (0, 3), (BLK, 3 + W))
        for k in range(1, 11):
            b += float(GW[k]) * lax.slice(padded, (0, 3 + k), (BLK, 3 + k + W))
        zz = (b - 0.5) * 50.0
        out_ref[...] = 1.0 / (1.0 + jnp.exp(-zz))


@functools.partial(jax.jit, static_argnames=("interpret",))
def _blur_call(canvas, interpret=False):
    out = jax.ShapeDtypeStruct((H, W), jnp.float32)
    return pl.pallas_call(
        _blur_body,
        grid=(GRID,),
        in_specs=[pl.BlockSpec((HC, W), lambda i: (0, 0))],
        out_specs=[pl.BlockSpec((BLK, W), lambda i: (i, 0))] * 4,
        out_shape=[out, out, out, out],
        interpret=interpret,
    )(canvas)


def _make_canvas(ep):
    x0 = ep[:, 0, 0]
    x1 = ep[:, 0, 1]
    y0 = ep[:, 1, 0]
    y1 = ep[:, 1, 1]
    t = jnp.linspace(0.0, 1.0, L, dtype=jnp.float32)[None, :]
    px = jnp.round(x0[:, None] + t * (x1 - x0)[:, None]).astype(jnp.int32)
    py = jnp.round(y0[:, None] + t * (y1 - y0)[:, None]).astype(jnp.int32)
    px = jnp.clip(px, 0, W - 1)
    py = jnp.clip(py, 0, H - 1)
    canvas = jnp.zeros((HC, W), jnp.float32)
    canvas = canvas.at[py.reshape(-1) + PAD, px.reshape(-1)].set(1.0)
    return canvas


def kernel(edge_params, velocities, iter_idx):
    ep = jnp.round(edge_params)
    ep = jnp.stack(
        [jnp.clip(ep[:, 0, :], 0.0, W - 1.0), jnp.clip(ep[:, 1, :], 0.0, H - 1.0)],
        axis=1,
    )
    canvas = _make_canvas(ep)
    mask, nom, mx, mn = _blur_call(canvas)
    return (mask, nom, mx, mn, ep)
